# Initial kernel scaffold; baseline (speedup 1.0000x reference)
import functools

import jax
import jax.numpy as jnp
from jax import lax
from jax.experimental import pallas as pl
from jax.experimental.pallas import tpu as pltpu
from jax.experimental.pallas import tpu_sc as plsc

N = 10000
E = 320000
D = 128

NTILES = 32
NSUB = 16
C = 128
KCH = 80
EPAD = NTILES * KCH * C
NP = 10240
RPT = NP // NSUB
BR = 256


def _mesh():
    return plsc.VectorSubcoreMesh(core_axis_name="c", subcore_axis_name="s")


def _deg_body(dsts_hbm, ones_hbm, zer_hbm, out_hbm, dst_v, ones_v, acc, sem):
    c = lax.axis_index("c")
    s = lax.axis_index("s")
    gid = c * NSUB + s
    r0 = s * RPT
    pltpu.sync_copy(zer_hbm, acc.at[pl.ds(r0, RPT)])
    pltpu.sync_copy(ones_hbm, ones_v)
    pltpu.sync_copy(dsts_hbm.at[gid], dst_v)
    plsc.subcore_barrier()

    def body(j, carry):
        pltpu.sync_copy(ones_v, acc.at[dst_v.at[j]], add=True)
        return carry

    lax.fori_loop(0, KCH, body, 0)
    plsc.subcore_barrier()
    pltpu.sync_copy(acc.at[pl.ds(r0, RPT)], out_hbm.at[c, pl.ds(r0, RPT)])


def _deg_call(dsts, ones_rows, zer_rows):
    f = pl.kernel(
        _deg_body,
        out_type=jax.ShapeDtypeStruct((2, NP, 16), jnp.float32),
        mesh=_mesh(),
        scratch_types=[
            pltpu.VMEM((KCH, C), jnp.int32),
            pltpu.VMEM((C, 16), jnp.float32),
            pltpu.VMEM_SHARED((NP, 16), jnp.float32),
            pltpu.SemaphoreType.DMA,
        ],
    )
    return f(dsts, ones_rows, zer_rows)


def _seg_body(g_hbm, srcs_hbm, dsts_hbm, zer_hbm, out_hbm,
              src_v, dst_v, buf, acc, sem):
    c = lax.axis_index("c")
    s = lax.axis_index("s")
    gid = c * NSUB + s
    r0 = s * RPT
    pltpu.sync_copy(zer_hbm, acc.at[pl.ds(r0, RPT)])
    pltpu.sync_copy(srcs_hbm.at[gid], src_v)
    pltpu.sync_copy(dsts_hbm.at[gid], dst_v)
    plsc.subcore_barrier()

    def body(j, carry):
        pltpu.async_copy(g_hbm.at[src_v.at[j]], buf, sem).wait()
        pltpu.sync_copy(buf, acc.at[dst_v.at[j]], add=True)
        return carry

    lax.fori_loop(0, KCH, body, 0)
    plsc.subcore_barrier()
    pltpu.sync_copy(acc.at[pl.ds(r0, RPT)], out_hbm.at[c, pl.ds(r0, RPT)])


def _seg_call(g, srcs, dsts, zer_rows):
    f = pl.kernel(
        _seg_body,
        out_type=jax.ShapeDtypeStruct((2, NP, D), jnp.float32),
        mesh=_mesh(),
        scratch_types=[
            pltpu.VMEM((KCH, C), jnp.int32),
            pltpu.VMEM((KCH, C), jnp.int32),
            pltpu.VMEM((C, D), jnp.float32),
            pltpu.VMEM_SHARED((NP, D), jnp.float32),
            pltpu.SemaphoreType.DMA,
        ],
    )
    return f(g, srcs, dsts, zer_rows)


def _dinv_of(degp):
    deg = degp[0, :, 0:1] + degp[1, :, 0:1] + 1.0
    return lax.rsqrt(deg)


def _k1_body(x_ref, w_ref, degp_ref, o_ref):
    dinv = _dinv_of(degp_ref[...])
    o_ref[...] = dinv * jnp.dot(x_ref[...], w_ref[...],
                                preferred_element_type=jnp.float32)


def _k1_call(xp, W1, degp):
    return pl.pallas_call(
        _k1_body,
        grid=(NP // BR,),
        in_specs=[
            pl.BlockSpec((BR, D), lambda i: (i, 0)),
            pl.BlockSpec((D, D), lambda i: (0, 0)),
            pl.BlockSpec((2, BR, 16), lambda i: (0, i, 0)),
        ],
        out_specs=pl.BlockSpec((BR, D), lambda i: (i, 0)),
        out_shape=jax.ShapeDtypeStruct((NP, D), jnp.float32),
    )(xp, W1, degp)


def _k2_body(part_ref, g1_ref, degp_ref, b_ref, w_ref, o_ref):
    i = pl.program_id(0)
    dinv = _dinv_of(degp_ref[...])
    p = part_ref[...]
    ssum = p[0] + p[1] + g1_ref[...]
    h = jnp.maximum(dinv * ssum + b_ref[...], 0.0)
    rows = i * BR + lax.broadcasted_iota(jnp.int32, (BR, 1), 0)
    h = jnp.where(rows < N, h, 0.0)
    o_ref[...] = dinv * jnp.dot(h, w_ref[...],
                                preferred_element_type=jnp.float32)


def _k2_call(part, g1, degp, b, W):
    return pl.pallas_call(
        _k2_body,
        grid=(NP // BR,),
        in_specs=[
            pl.BlockSpec((2, BR, D), lambda i: (0, i, 0)),
            pl.BlockSpec((BR, D), lambda i: (i, 0)),
            pl.BlockSpec((2, BR, 16), lambda i: (0, i, 0)),
            pl.BlockSpec((1, D), lambda i: (0, 0)),
            pl.BlockSpec((D, D), lambda i: (0, 0)),
        ],
        out_specs=pl.BlockSpec((BR, D), lambda i: (i, 0)),
        out_shape=jax.ShapeDtypeStruct((NP, D), jnp.float32),
    )(part, g1, degp, b, W)


def _k3_body(part_ref, g2_ref, degp_ref, b_ref, lw_ref, lb_ref, o_ref):
    dinv = _dinv_of(degp_ref[...])
    p = part_ref[...]
    ssum = p[0] + p[1] + g2_ref[...]
    h = jnp.maximum(dinv * ssum + b_ref[...], 0.0)
    o_ref[...] = jnp.sum(h * lw_ref[...], axis=1, keepdims=True) + lb_ref[0, 0]


def _k3_call(part, g2, degp, b, lw_row, lb):
    return pl.pallas_call(
        _k3_body,
        grid=(NP // BR,),
        in_specs=[
            pl.BlockSpec((2, BR, D), lambda i: (0, i, 0)),
            pl.BlockSpec((BR, D), lambda i: (i, 0)),
            pl.BlockSpec((2, BR, 16), lambda i: (0, i, 0)),
            pl.BlockSpec((1, D), lambda i: (0, 0)),
            pl.BlockSpec((1, D), lambda i: (0, 0)),
            pl.BlockSpec((1, 1), lambda i: (0, 0)),
        ],
        out_specs=pl.BlockSpec((BR, 1), lambda i: (i, 0)),
        out_shape=jax.ShapeDtypeStruct((NP, 1), jnp.float32),
    )(part, g2, degp, b, lw_row, lb)


def kernel(x, edge_index, W1, b1, W2, b2, lin_W, lin_b):
    src = edge_index[0]
    dst = edge_index[1]
    fill = jnp.full((EPAD - E,), N, dtype=jnp.int32)
    srcs = jnp.concatenate([src, fill]).reshape(NTILES, KCH, C)
    dsts = jnp.concatenate([dst, fill]).reshape(NTILES, KCH, C)
    xp = jnp.pad(x, ((0, NP - N), (0, 0)))
    ones_rows = jnp.ones((C, 16), jnp.float32)
    zer16 = jnp.zeros((RPT, 16), jnp.float32)
    zer128 = jnp.zeros((RPT, D), jnp.float32)

    degp = _deg_call(dsts, ones_rows, zer16)
    g1 = _k1_call(xp, W1, degp)
    part1 = _seg_call(g1, srcs, dsts, zer128)
    g2 = _k2_call(part1, g1, degp, b1.reshape(1, D), W2)
    part2 = _seg_call(g2, srcs, dsts, zer128)
    o = _k3_call(part2, g2, degp, b2.reshape(1, D),
                 lin_W.reshape(1, D), lin_b.reshape(1, 1))
    return o[:N, 0]


# speedup vs baseline: 6.6070x; 6.6070x over previous
import functools

import jax
import jax.numpy as jnp
from jax import lax
from jax.experimental import pallas as pl
from jax.experimental.pallas import tpu as pltpu
from jax.experimental.pallas import tpu_sc as plsc

N = 10000
E = 320000
D = 128

NTILES = 32
NSUB = 16
C = 128
KCH = 80
EPAD = NTILES * KCH * C
NP = 10240
RPT = NP // NSUB
BR = 256


def _mesh():
    return plsc.VectorSubcoreMesh(core_axis_name="c", subcore_axis_name="s",
                                  num_cores=2, num_subcores=NSUB)


def _deg_body(dsts_hbm, ones_hbm, zer_hbm, out_hbm, dst_c, ones_v, acc, sem):
    c = lax.axis_index("c")
    s = lax.axis_index("s")
    gid = c * NSUB + s
    r0 = s * RPT
    pltpu.sync_copy(zer_hbm, acc.at[pl.ds(r0, RPT)])
    pltpu.sync_copy(ones_hbm, ones_v)
    plsc.subcore_barrier()

    def body(j, carry):
        pltpu.sync_copy(dsts_hbm.at[gid, j], dst_c)
        pltpu.sync_copy(ones_v, acc.at[dst_c], add=True)
        return carry

    lax.fori_loop(0, KCH, body, 0)
    plsc.subcore_barrier()
    pltpu.sync_copy(acc.at[pl.ds(r0, RPT)], out_hbm.at[c, pl.ds(r0, RPT)])


def _deg_call(dsts, ones_rows, zer_rows):
    f = pl.kernel(
        _deg_body,
        out_type=jax.ShapeDtypeStruct((2, NP, D), jnp.float32),
        mesh=_mesh(),
        scratch_types=[
            pltpu.VMEM((C,), jnp.int32),
            pltpu.VMEM((C, D), jnp.float32),
            pltpu.VMEM_SHARED((NP, D), jnp.float32),
            pltpu.SemaphoreType.DMA,
        ],
    )
    return f(dsts, ones_rows, zer_rows)


def _seg_body(g_hbm, srcs_hbm, dsts_hbm, zer_hbm, out_hbm,
              src_c, dst_c, buf, acc, sem):
    c = lax.axis_index("c")
    s = lax.axis_index("s")
    gid = c * NSUB + s
    r0 = s * RPT
    pltpu.sync_copy(zer_hbm, acc.at[pl.ds(r0, RPT)])
    plsc.subcore_barrier()

    def body(j, carry):
        pltpu.sync_copy(srcs_hbm.at[gid, j], src_c)
        pltpu.sync_copy(dsts_hbm.at[gid, j], dst_c)
        pltpu.async_copy(g_hbm.at[src_c], buf, sem).wait()
        pltpu.sync_copy(buf, acc.at[dst_c], add=True)
        return carry

    lax.fori_loop(0, KCH, body, 0)
    plsc.subcore_barrier()
    pltpu.sync_copy(acc.at[pl.ds(r0, RPT)], out_hbm.at[c, pl.ds(r0, RPT)])


def _seg_call(g, srcs, dsts, zer_rows):
    f = pl.kernel(
        _seg_body,
        out_type=jax.ShapeDtypeStruct((2, NP, D), jnp.float32),
        mesh=_mesh(),
        scratch_types=[
            pltpu.VMEM((C,), jnp.int32),
            pltpu.VMEM((C,), jnp.int32),
            pltpu.VMEM((C, D), jnp.float32),
            pltpu.VMEM_SHARED((NP, D), jnp.float32),
            pltpu.SemaphoreType.DMA,
        ],
    )
    return f(g, srcs, dsts, zer_rows)


def _dinv_of(degp):
    deg = degp[0, :, 0:1] + degp[1, :, 0:1] + 1.0
    return lax.rsqrt(deg)


def _k1_body(x_ref, w_ref, degp_ref, o_ref):
    dinv = _dinv_of(degp_ref[...])
    o_ref[...] = dinv * jnp.dot(x_ref[...], w_ref[...],
                                preferred_element_type=jnp.float32)


def _k1_call(xp, W1, degp):
    return pl.pallas_call(
        _k1_body,
        grid=(NP // BR,),
        in_specs=[
            pl.BlockSpec((BR, D), lambda i: (i, 0)),
            pl.BlockSpec((D, D), lambda i: (0, 0)),
            pl.BlockSpec((2, BR, D), lambda i: (0, i, 0)),
        ],
        out_specs=pl.BlockSpec((BR, D), lambda i: (i, 0)),
        out_shape=jax.ShapeDtypeStruct((NP, D), jnp.float32),
    )(xp, W1, degp)


def _k2_body(part_ref, g1_ref, degp_ref, b_ref, w_ref, o_ref):
    i = pl.program_id(0)
    dinv = _dinv_of(degp_ref[...])
    p = part_ref[...]
    ssum = p[0] + p[1] + g1_ref[...]
    h = jnp.maximum(dinv * ssum + b_ref[...], 0.0)
    rows = i * BR + lax.broadcasted_iota(jnp.int32, (BR, 1), 0)
    h = jnp.where(rows < N, h, 0.0)
    o_ref[...] = dinv * jnp.dot(h, w_ref[...],
                                preferred_element_type=jnp.float32)


def _k2_call(part, g1, degp, b, W):
    return pl.pallas_call(
        _k2_body,
        grid=(NP // BR,),
        in_specs=[
            pl.BlockSpec((2, BR, D), lambda i: (0, i, 0)),
            pl.BlockSpec((BR, D), lambda i: (i, 0)),
            pl.BlockSpec((2, BR, D), lambda i: (0, i, 0)),
            pl.BlockSpec((1, D), lambda i: (0, 0)),
            pl.BlockSpec((D, D), lambda i: (0, 0)),
        ],
        out_specs=pl.BlockSpec((BR, D), lambda i: (i, 0)),
        out_shape=jax.ShapeDtypeStruct((NP, D), jnp.float32),
    )(part, g1, degp, b, W)


def _k3_body(part_ref, g2_ref, degp_ref, b_ref, lw_ref, lb_ref, o_ref):
    dinv = _dinv_of(degp_ref[...])
    p = part_ref[...]
    ssum = p[0] + p[1] + g2_ref[...]
    h = jnp.maximum(dinv * ssum + b_ref[...], 0.0)
    o_ref[...] = jnp.sum(h * lw_ref[...], axis=1, keepdims=True) + lb_ref[0, 0]


def _k3_call(part, g2, degp, b, lw_row, lb):
    return pl.pallas_call(
        _k3_body,
        grid=(NP // BR,),
        in_specs=[
            pl.BlockSpec((2, BR, D), lambda i: (0, i, 0)),
            pl.BlockSpec((BR, D), lambda i: (i, 0)),
            pl.BlockSpec((2, BR, D), lambda i: (0, i, 0)),
            pl.BlockSpec((1, D), lambda i: (0, 0)),
            pl.BlockSpec((1, D), lambda i: (0, 0)),
            pl.BlockSpec((1, 1), lambda i: (0, 0)),
        ],
        out_specs=pl.BlockSpec((BR, 1), lambda i: (i, 0)),
        out_shape=jax.ShapeDtypeStruct((NP, 1), jnp.float32),
    )(part, g2, degp, b, lw_row, lb)


def kernel(x, edge_index, W1, b1, W2, b2, lin_W, lin_b):
    src = edge_index[0]
    dst = edge_index[1]
    fill = jnp.full((EPAD - E,), N, dtype=jnp.int32)
    srcs = jnp.concatenate([src, fill]).reshape(NTILES, KCH, C)
    dsts = jnp.concatenate([dst, fill]).reshape(NTILES, KCH, C)
    xp = jnp.pad(x, ((0, NP - N), (0, 0)))
    ones_rows = jnp.ones((C, D), jnp.float32)
    zer128 = jnp.zeros((RPT, D), jnp.float32)

    degp = _deg_call(dsts, ones_rows, zer128)
    g1 = _k1_call(xp, W1, degp)
    part1 = _seg_call(g1, srcs, dsts, zer128)
    g2 = _k2_call(part1, g1, degp, b1.reshape(1, D), W2)
    part2 = _seg_call(g2, srcs, dsts, zer128)
    o = _k3_call(part2, g2, degp, b2.reshape(1, D),
                 lin_W.reshape(1, D), lin_b.reshape(1, 1))
    return o[:N, 0]


# speedup vs baseline: 8.1721x; 1.2369x over previous
import functools

import jax
import jax.numpy as jnp
from jax import lax
from jax.experimental import pallas as pl
from jax.experimental.pallas import tpu as pltpu
from jax.experimental.pallas import tpu_sc as plsc

N = 10000
E = 320000
D = 128

NTILES = 32
NSUB = 16
C = 128
KCH = 80
EPAD = NTILES * KCH * C
NP = 10240
RPT = NP // NSUB
BR = 256


def _mesh():
    return plsc.VectorSubcoreMesh(core_axis_name="c", subcore_axis_name="s",
                                  num_cores=2, num_subcores=NSUB)


def _deg_body(dsts_hbm, ones_hbm, zer_hbm, out_hbm, dst0, dst1, ones_v, acc,
              sem0, sem1):
    c = lax.axis_index("c")
    s = lax.axis_index("s")
    gid = c * NSUB + s
    r0 = s * RPT
    pltpu.sync_copy(zer_hbm, acc.at[pl.ds(r0, RPT)])
    pltpu.sync_copy(ones_hbm, ones_v)
    plsc.subcore_barrier()
    pltpu.async_copy(dsts_hbm.at[gid, 0], dst0, sem0)

    def body(jj, carry):
        j0 = 2 * jj
        pltpu.async_copy(dsts_hbm.at[gid, j0 + 1], dst1, sem1)
        pltpu.make_async_copy(dsts_hbm.at[gid, j0], dst0, sem0).wait()
        pltpu.sync_copy(ones_v, acc.at[dst0], add=True)

        @pl.when(j0 + 2 < KCH)
        def _():
            pltpu.async_copy(dsts_hbm.at[gid, j0 + 2], dst0, sem0)

        pltpu.make_async_copy(dsts_hbm.at[gid, j0 + 1], dst1, sem1).wait()
        pltpu.sync_copy(ones_v, acc.at[dst1], add=True)
        return carry

    lax.fori_loop(0, KCH // 2, body, 0)
    plsc.subcore_barrier()
    pltpu.sync_copy(acc.at[pl.ds(r0, RPT)], out_hbm.at[c, pl.ds(r0, RPT)])


def _deg_call(dsts, ones_rows, zer_rows):
    f = pl.kernel(
        _deg_body,
        out_type=jax.ShapeDtypeStruct((2, NP, D), jnp.float32),
        mesh=_mesh(),
        scratch_types=[
            pltpu.VMEM((C,), jnp.int32),
            pltpu.VMEM((C,), jnp.int32),
            pltpu.VMEM((C, D), jnp.float32),
            pltpu.VMEM_SHARED((NP, D), jnp.float32),
            pltpu.SemaphoreType.DMA,
            pltpu.SemaphoreType.DMA,
        ],
    )
    return f(dsts, ones_rows, zer_rows)


def _seg_body(g_hbm, srcs_hbm, dsts_hbm, zer_hbm, out_hbm,
              src0, src1, dst0, dst1, buf0, buf1, acc, sem0, sem1):
    c = lax.axis_index("c")
    s = lax.axis_index("s")
    gid = c * NSUB + s
    r0 = s * RPT
    pltpu.sync_copy(zer_hbm, acc.at[pl.ds(r0, RPT)])
    plsc.subcore_barrier()
    pltpu.sync_copy(srcs_hbm.at[gid, 0], src0)
    pltpu.sync_copy(dsts_hbm.at[gid, 0], dst0)
    pltpu.async_copy(g_hbm.at[src0], buf0, sem0)

    def body(jj, carry):
        j0 = 2 * jj
        pltpu.sync_copy(srcs_hbm.at[gid, j0 + 1], src1)
        pltpu.sync_copy(dsts_hbm.at[gid, j0 + 1], dst1)
        pltpu.async_copy(g_hbm.at[src1], buf1, sem1)
        pltpu.make_async_copy(g_hbm.at[src0], buf0, sem0).wait()
        pltpu.sync_copy(buf0, acc.at[dst0], add=True)

        @pl.when(j0 + 2 < KCH)
        def _():
            pltpu.sync_copy(srcs_hbm.at[gid, j0 + 2], src0)
            pltpu.sync_copy(dsts_hbm.at[gid, j0 + 2], dst0)
            pltpu.async_copy(g_hbm.at[src0], buf0, sem0)

        pltpu.make_async_copy(g_hbm.at[src1], buf1, sem1).wait()
        pltpu.sync_copy(buf1, acc.at[dst1], add=True)
        return carry

    lax.fori_loop(0, KCH // 2, body, 0)
    plsc.subcore_barrier()
    pltpu.sync_copy(acc.at[pl.ds(r0, RPT)], out_hbm.at[c, pl.ds(r0, RPT)])


def _seg_call(g, srcs, dsts, zer_rows):
    f = pl.kernel(
        _seg_body,
        out_type=jax.ShapeDtypeStruct((2, NP, D), jnp.float32),
        mesh=_mesh(),
        scratch_types=[
            pltpu.VMEM((C,), jnp.int32),
            pltpu.VMEM((C,), jnp.int32),
            pltpu.VMEM((C,), jnp.int32),
            pltpu.VMEM((C,), jnp.int32),
            pltpu.VMEM((C, D), jnp.float32),
            pltpu.VMEM((C, D), jnp.float32),
            pltpu.VMEM_SHARED((NP, D), jnp.float32),
            pltpu.SemaphoreType.DMA,
            pltpu.SemaphoreType.DMA,
        ],
    )
    return f(g, srcs, dsts, zer_rows)


def _dinv_of(degp):
    deg = degp[0, :, 0:1] + degp[1, :, 0:1] + 1.0
    return lax.rsqrt(deg)


def _k1_body(x_ref, w_ref, degp_ref, o_ref):
    dinv = _dinv_of(degp_ref[...])
    o_ref[...] = dinv * jnp.dot(x_ref[...], w_ref[...],
                                preferred_element_type=jnp.float32)


def _k1_call(xp, W1, degp):
    return pl.pallas_call(
        _k1_body,
        grid=(NP // BR,),
        in_specs=[
            pl.BlockSpec((BR, D), lambda i: (i, 0)),
            pl.BlockSpec((D, D), lambda i: (0, 0)),
            pl.BlockSpec((2, BR, D), lambda i: (0, i, 0)),
        ],
        out_specs=pl.BlockSpec((BR, D), lambda i: (i, 0)),
        out_shape=jax.ShapeDtypeStruct((NP, D), jnp.float32),
    )(xp, W1, degp)


def _k2_body(part_ref, g1_ref, degp_ref, b_ref, w_ref, o_ref):
    i = pl.program_id(0)
    dinv = _dinv_of(degp_ref[...])
    p = part_ref[...]
    ssum = p[0] + p[1] + g1_ref[...]
    h = jnp.maximum(dinv * ssum + b_ref[...], 0.0)
    rows = i * BR + lax.broadcasted_iota(jnp.int32, (BR, 1), 0)
    h = jnp.where(rows < N, h, 0.0)
    o_ref[...] = dinv * jnp.dot(h, w_ref[...],
                                preferred_element_type=jnp.float32)


def _k2_call(part, g1, degp, b, W):
    return pl.pallas_call(
        _k2_body,
        grid=(NP // BR,),
        in_specs=[
            pl.BlockSpec((2, BR, D), lambda i: (0, i, 0)),
            pl.BlockSpec((BR, D), lambda i: (i, 0)),
            pl.BlockSpec((2, BR, D), lambda i: (0, i, 0)),
            pl.BlockSpec((1, D), lambda i: (0, 0)),
            pl.BlockSpec((D, D), lambda i: (0, 0)),
        ],
        out_specs=pl.BlockSpec((BR, D), lambda i: (i, 0)),
        out_shape=jax.ShapeDtypeStruct((NP, D), jnp.float32),
    )(part, g1, degp, b, W)


def _k3_body(part_ref, g2_ref, degp_ref, b_ref, lw_ref, lb_ref, o_ref):
    dinv = _dinv_of(degp_ref[...])
    p = part_ref[...]
    ssum = p[0] + p[1] + g2_ref[...]
    h = jnp.maximum(dinv * ssum + b_ref[...], 0.0)
    o_ref[...] = jnp.sum(h * lw_ref[...], axis=1, keepdims=True) + lb_ref[0, 0]


def _k3_call(part, g2, degp, b, lw_row, lb):
    return pl.pallas_call(
        _k3_body,
        grid=(NP // BR,),
        in_specs=[
            pl.BlockSpec((2, BR, D), lambda i: (0, i, 0)),
            pl.BlockSpec((BR, D), lambda i: (i, 0)),
            pl.BlockSpec((2, BR, D), lambda i: (0, i, 0)),
            pl.BlockSpec((1, D), lambda i: (0, 0)),
            pl.BlockSpec((1, D), lambda i: (0, 0)),
            pl.BlockSpec((1, 1), lambda i: (0, 0)),
        ],
        out_specs=pl.BlockSpec((BR, 1), lambda i: (i, 0)),
        out_shape=jax.ShapeDtypeStruct((NP, 1), jnp.float32),
    )(part, g2, degp, b, lw_row, lb)


def kernel(x, edge_index, W1, b1, W2, b2, lin_W, lin_b):
    src = edge_index[0]
    dst = edge_index[1]
    fill = jnp.full((EPAD - E,), N, dtype=jnp.int32)
    srcs = jnp.concatenate([src, fill]).reshape(NTILES, KCH, C)
    dsts = jnp.concatenate([dst, fill]).reshape(NTILES, KCH, C)
    xp = jnp.pad(x, ((0, NP - N), (0, 0)))
    ones_rows = jnp.ones((C, D), jnp.float32)
    zer128 = jnp.zeros((RPT, D), jnp.float32)

    degp = _deg_call(dsts, ones_rows, zer128)
    g1 = _k1_call(xp, W1, degp)
    part1 = _seg_call(g1, srcs, dsts, zer128)
    g2 = _k2_call(part1, g1, degp, b1.reshape(1, D), W2)
    part2 = _seg_call(g2, srcs, dsts, zer128)
    o = _k3_call(part2, g2, degp, b2.reshape(1, D),
                 lin_W.reshape(1, D), lin_b.reshape(1, 1))
    return o[:N, 0]
